# baseline (speedup 1.0000x reference)
import functools

import jax
import jax.numpy as jnp
from jax.experimental import pallas as pl
from jax.experimental.pallas import tpu as pltpu
from jax.experimental.pallas import tpu_sc as plsc

R = 128
D = 128
STRIDE = 128

_mesh = plsc.ScalarSubcoreMesh(axis_name="c", num_cores=1)


@functools.partial(
    pl.kernel,
    out_type=jax.ShapeDtypeStruct((R, 1, D), jnp.float32),
    mesh=_mesh,
)
def _gather_rows(x3_hbm, out_hbm):
    pltpu.sync_copy(x3_hbm.at[:, pl.ds(0, 1), :], out_hbm)


def kernel(mamdani_output):
    x3 = mamdani_output.reshape(R, STRIDE, D)
    return _gather_rows(x3)


# speedup vs baseline: 1.0130x; 1.0130x over previous
import functools

import jax
import jax.numpy as jnp
from jax import lax
from jax.experimental import pallas as pl
from jax.experimental.pallas import tpu as pltpu
from jax.experimental.pallas import tpu_sc as plsc

R = 128
D = 128
STRIDE = 128
ROWS_PER_WORKER = 16
NWORK = R // ROWS_PER_WORKER

_mesh = plsc.VectorSubcoreMesh(
    core_axis_name="c", subcore_axis_name="s", num_cores=1, num_subcores=NWORK
)


@functools.partial(
    pl.kernel,
    out_type=jax.ShapeDtypeStruct((R, D), jnp.float32),
    mesh=_mesh,
    scratch_types=[
        pltpu.VMEM((ROWS_PER_WORKER, D), jnp.float32),
        pltpu.SemaphoreType.DMA,
    ],
)
def _gather_rows(x_hbm, out_hbm, rows_v, sem):
    wid = lax.axis_index("s")
    row0 = wid * ROWS_PER_WORKER
    idx = (lax.iota(jnp.int32, ROWS_PER_WORKER) + row0) * STRIDE
    pltpu.async_copy(x_hbm.at[idx], rows_v, sem).wait()
    pltpu.sync_copy(rows_v, out_hbm.at[pl.ds(row0, ROWS_PER_WORKER)])


def kernel(mamdani_output):
    gathered = _gather_rows(mamdani_output)
    return gathered[:, None, :]
